# baseline (speedup 1.0000x reference)
import functools

import jax
import jax.numpy as jnp
from jax import lax
from jax.experimental import pallas as pl
from jax.experimental.pallas import tpu as pltpu
from jax.experimental.pallas import tpu_sc as plsc

N = 10000
E = 320000
D = 128
HID = 128
NQ = 3
NORM = 100.0
MU = 0.5

NBLK = 2000
EBLK = 2000


def _silu(x):
    return x * jax.nn.sigmoid(x)


def _pq_body(h_ref, w_ref, p_ref, q_ref):
    h = h_ref[...]
    p_ref[...] = jnp.dot(h, w_ref[:D, :], preferred_element_type=jnp.float32)
    q_ref[...] = jnp.dot(h, w_ref[D:, :], preferred_element_type=jnp.float32)


def _pq(h, ew1):
    return pl.pallas_call(
        _pq_body,
        grid=(N // NBLK,),
        in_specs=[
            pl.BlockSpec((NBLK, D), lambda i: (i, 0)),
            pl.BlockSpec((2 * D, HID), lambda i: (0, 0)),
        ],
        out_specs=[
            pl.BlockSpec((NBLK, HID), lambda i: (i, 0)),
            pl.BlockSpec((NBLK, HID), lambda i: (i, 0)),
        ],
        out_shape=[
            jax.ShapeDtypeStruct((N, HID), jnp.float32),
            jax.ShapeDtypeStruct((N, HID), jnp.float32),
        ],
    )(h, ew1)


_NW = 32
_K = 5
_EC = E // _K
_EW = _EC // _NW
_G = 80
_NCH = _EW // _G

_VMESH = plsc.VectorSubcoreMesh(core_axis_name="c", subcore_axis_name="s")


def _sc_gather_body(ofs, p_hbm, q_hbm, row_hbm, col_hbm, r_hbm,
                    ri_v, ci_v, bp_v, bq_v, bo_v, gsem0, gsem1, ssem0, ssem1):
    gsem = (gsem0, gsem1)
    ssem = (ssem0, ssem1)
    wid = lax.axis_index("c") * 16 + lax.axis_index("s")
    gbase = ofs + wid * _EW
    base = wid * _EW
    pltpu.sync_copy(row_hbm.at[pl.ds(gbase, _EW)], ri_v)
    pltpu.sync_copy(col_hbm.at[pl.ds(gbase, _EW)], ci_v)

    def start_gather(c, slot):
        pltpu.async_copy(p_hbm.at[ri_v.at[pl.ds(c * _G, _G)]],
                         bp_v.at[slot], gsem[slot])
        pltpu.async_copy(q_hbm.at[ci_v.at[pl.ds(c * _G, _G)]],
                         bq_v.at[slot], gsem[slot])

    def wait_gather(slot):
        pltpu.make_async_copy(p_hbm.at[ri_v.at[pl.ds(0, _G)]],
                              bp_v.at[slot], gsem[slot]).wait()
        pltpu.make_async_copy(q_hbm.at[ci_v.at[pl.ds(0, _G)]],
                              bq_v.at[slot], gsem[slot]).wait()

    def wait_store(slot):
        pltpu.make_async_copy(bo_v.at[slot], r_hbm.at[pl.ds(base, _G)],
                              ssem[slot]).wait()

    def add_store(c, slot):
        bp = bp_v.at[slot]
        bq = bq_v.at[slot]
        bo = bo_v.at[slot]

        @pl.loop(0, _G)
        def _(r):
            for c0 in range(0, HID, 16):
                bo[r, pl.ds(c0, 16)] = (bp[r, pl.ds(c0, 16)]
                                        + bq[r, pl.ds(c0, 16)])

        pltpu.async_copy(bo, r_hbm.at[pl.ds(base + c * _G, _G)], ssem[slot])

    start_gather(0, 0)
    start_gather(1, 1)
    _GMAIN = (_NCH // 2) * 2

    @pl.loop(0, _GMAIN, step=2)
    def _(i):
        for slot in (0, 1):
            c = i + slot
            wait_gather(slot)

            @pl.when(c >= 2)
            def _():
                wait_store(slot)

            add_store(c, slot)
            nxt = jnp.minimum(c + 2, _NCH - 1)
            start_gather(nxt, slot)

    wait_gather(0)
    wait_store(0)
    add_store(_NCH - 1, 0)
    wait_gather(1)
    wait_store(0)
    wait_store(1)


def _sc_gather_add(p, q, row, col, k):
    f = pl.kernel(
        functools.partial(_sc_gather_body, k * _EC),
        out_type=jax.ShapeDtypeStruct((_EC, HID), jnp.float32),
        mesh=_VMESH,
        scratch_types=[
            pltpu.VMEM((_EW,), jnp.int32),
            pltpu.VMEM((_EW,), jnp.int32),
            pltpu.VMEM((2, _G, HID), jnp.float32),
            pltpu.VMEM((2, _G, HID), jnp.float32),
            pltpu.VMEM((2, _G, HID), jnp.float32),
            pltpu.SemaphoreType.DMA,
            pltpu.SemaphoreType.DMA,
            pltpu.SemaphoreType.DMA,
            pltpu.SemaphoreType.DMA,
        ],
        name=f"sc_gather_{k}",
    )
    return f(p, q, row, col)


_TROWS = 624
_ZB = 48
_TAIL0 = 16 * _TROWS
_TAILN = N - _TAIL0


_SS = 4
_LOOK = 2
_PCH = _EW // _G
_PMAIN = (_PCH // _SS) * _SS


def _sc_segsum_body(ofs, mij_hbm, row_hbm, out_hbm, ri_v, mb_v, zb_v, acc_sh,
                    lsem, asem):
    cid = lax.axis_index("c")
    sid = lax.axis_index("s")
    base = (cid * 16 + sid) * _EW
    pltpu.async_copy(row_hbm.at[pl.ds(ofs + base, _EW)], ri_v,
                     lsem.at[_SS - 1])

    @pl.loop(0, _ZB)
    def _(r):
        for c0 in range(0, HID, 16):
            zb_v[r, pl.ds(c0, 16)] = jnp.zeros((16,), jnp.float32)

    @pl.loop(0, _TROWS // _ZB)
    def _(i):
        pltpu.sync_copy(zb_v, acc_sh.at[pl.ds(sid * _TROWS + i * _ZB, _ZB)])

    @pl.when(sid == 0)
    def _():
        pltpu.sync_copy(zb_v.at[pl.ds(0, _TAILN)],
                        acc_sh.at[pl.ds(_TAIL0, _TAILN)])

    pltpu.make_async_copy(row_hbm.at[pl.ds(ofs + base, _EW)], ri_v,
                          lsem.at[_SS - 1]).wait()
    plsc.subcore_barrier()

    def start_load(c, slot):
        pltpu.async_copy(mij_hbm.at[pl.ds(base + c * _G, _G)],
                         mb_v.at[slot], lsem.at[slot])

    def wait_load(slot):
        pltpu.make_async_copy(mij_hbm.at[pl.ds(base, _G)],
                              mb_v.at[slot], lsem.at[slot]).wait()

    def start_add(c, slot):
        pltpu.async_copy(mb_v.at[slot],
                         acc_sh.at[ri_v.at[pl.ds(c * _G, _G)]],
                         asem.at[slot], add=True)

    def wait_add(slot):
        pltpu.make_async_copy(mb_v.at[slot],
                              acc_sh.at[ri_v.at[pl.ds(0, _G)]],
                              asem.at[slot]).wait()

    for c in range(_LOOK):
        start_load(c, c)

    @pl.loop(0, _PMAIN, step=_SS)
    def _(i):
        for b in range(_SS):
            c = i + b
            nslot = (b + _LOOK) % _SS
            wait_load(b)
            start_add(c, b)

            @pl.when(c >= _LOOK)
            def _():
                wait_add(nslot)

            @pl.when(c + _LOOK < _PCH)
            def _():
                start_load(c + _LOOK, nslot)

    for c in range(_PMAIN, _PCH):
        wait_load(c % _SS)
        start_add(c, c % _SS)
    for c in range(_PMAIN - _LOOK, _PCH):
        wait_add(c % _SS)

    plsc.subcore_barrier()

    @pl.loop(0, _TROWS // _ZB)
    def _(i):
        r0 = sid * _TROWS + i * _ZB
        pltpu.sync_copy(acc_sh.at[pl.ds(r0, _ZB)],
                        out_hbm.at[cid, pl.ds(r0, _ZB)])

    @pl.when(sid == 0)
    def _():
        pltpu.sync_copy(acc_sh.at[pl.ds(_TAIL0, _TAILN)],
                        out_hbm.at[cid, pl.ds(_TAIL0, _TAILN)])


def _sc_segsum(mij_k, row, k):
    f = pl.kernel(
        functools.partial(_sc_segsum_body, k * _EC),
        out_type=jax.ShapeDtypeStruct((2, N, HID), jnp.float32),
        mesh=_VMESH,
        scratch_types=[
            pltpu.VMEM((_EW,), jnp.int32),
            pltpu.VMEM((_SS, _G, HID), jnp.float32),
            pltpu.VMEM((_ZB, HID), jnp.float32),
            pltpu.VMEM_SHARED((N, HID), jnp.float32),
            pltpu.SemaphoreType.DMA((_SS,)),
            pltpu.SemaphoreType.DMA((_SS,)),
        ],
        name=f"sc_segsum_{k}",
    )
    return f(mij_k, row)


def _emlp_body(r_ref, b1_ref, w2_ref, b2_ref, m_ref):
    hid = _silu(r_ref[...] + b1_ref[...])
    m_ref[...] = _silu(
        jnp.dot(hid, w2_ref[...], preferred_element_type=jnp.float32)
        + b2_ref[...]
    )


def _emlp(r, eb1, ew2, eb2):
    return pl.pallas_call(
        _emlp_body,
        grid=(_EC // EBLK,),
        in_specs=[
            pl.BlockSpec((EBLK, HID), lambda i: (i, 0)),
            pl.BlockSpec((1, HID), lambda i: (0, 0)),
            pl.BlockSpec((HID, HID), lambda i: (0, 0)),
            pl.BlockSpec((1, HID), lambda i: (0, 0)),
        ],
        out_specs=pl.BlockSpec((EBLK, HID), lambda i: (i, 0)),
        out_shape=jax.ShapeDtypeStruct((_EC, HID), jnp.float32),
    )(r, eb1.reshape(1, HID), ew2, eb2.reshape(1, HID))


_P_ALPHA, _P_SB, _P_CB, _P_DELTA, _P_BETA = 0, 1, 2, 3, 4
_P_QB2 = 5
_P_PHI = 8


def _node_body(params_ref, h_ref, *rest):
    (pa_refs, (qw1_ref, qw2_ref, qb1_ref, pw1_ref, pb1_ref, pw2_ref, pb2_ref,
               out_ref)) = rest[:_K], rest[_K:]
    h = h_ref[...]
    acc = pa_refs[0][0] + pa_refs[0][1]
    for k in range(1, _K):
        acc = acc + pa_refs[k][0] + pa_refs[k][1]
    agg = acc * (1.0 / NORM)
    cat = jnp.concatenate([h, agg], axis=1)
    hq = _silu(jnp.dot(cat, qw1_ref[...], preferred_element_type=jnp.float32)
               + qb1_ref[...])
    qin_t = lax.dot_general(qw2_ref[...], hq,
                            (((0,), (1,)), ((), ())),
                            preferred_element_type=jnp.float32)

    alpha = params_ref[_P_ALPHA]
    sb = params_ref[_P_SB]
    cb = params_ref[_P_CB]
    delta = params_ref[_P_DELTA]
    beta = params_ref[_P_BETA]

    q = [qin_t[k:k + 1, :] + params_ref[_P_QB2 + k] for k in range(NQ)]
    sa = [jnp.sin(alpha * qk) for qk in q]
    ca = [jnp.cos(alpha * qk) for qk in q]
    _pairidx = {(0, 1): 0, (0, 2): 1, (1, 2): 2}

    zrows = []
    for k in range(NQ):
        fr, fi = None, None
        for j in range(NQ):
            if j == k:
                continue
            pi = _pairidx[(min(k, j), max(k, j))]
            cp = params_ref[_P_PHI + 2 * pi]
            sp = params_ref[_P_PHI + 2 * pi + 1]
            gr = cp
            gi = -sp * ca[j]
            if fr is None:
                fr, fi = jnp.full_like(ca[j], gr), gi
            else:
                fr, fi = fr * gr - fi * gi, fr * gi + fi * gr
        qk = q[k]
        q2 = qk * qk
        d1 = delta * (1.0 - MU * q2)
        c = beta + delta * q2
        sc, cc = jnp.sin(c), jnp.cos(c)
        sd1, cd1 = jnp.sin(d1), jnp.cos(d1)
        rx = sa[k] * fr
        ry = -sa[k] * fi
        rz = ca[k]
        mx = sc * sd1
        my = sc * cd1 * cb + cc * sb
        mz = -sc * cd1 * sb + cc * cb
        zrows.append(rx * mx + ry * my + rz * mz)
    qout_t = jnp.concatenate(zrows, axis=0)

    hp_pre = (jnp.dot(cat, pw1_ref[:2 * D, :], preferred_element_type=jnp.float32)
              + lax.dot_general(qout_t, pw1_ref[2 * D:2 * D + NQ, :],
                                (((0,), (0,)), ((), ())),
                                preferred_element_type=jnp.float32)
              + pb1_ref[...])
    hp = _silu(hp_pre)
    out_ref[...] = h + jnp.dot(hp, pw2_ref[...],
                               preferred_element_type=jnp.float32) + pb2_ref[...]


def _node_stage(h, parts, qw1, qb1, qw2, qb2, pw1, pb1, pw2, pb2,
                alpha, beta, gamma, delta, Lam):
    phi = gamma * (Lam + Lam.T) / 2.0
    params = jnp.concatenate([
        jnp.stack([alpha, jnp.sin(beta), jnp.cos(beta), delta, beta]),
        qb2,
        jnp.stack([jnp.cos(phi[0, 1]), jnp.sin(phi[0, 1]),
                   jnp.cos(phi[0, 2]), jnp.sin(phi[0, 2]),
                   jnp.cos(phi[1, 2]), jnp.sin(phi[1, 2])]),
    ]).astype(jnp.float32)
    blk = lambda shape: pl.BlockSpec(shape, lambda i: tuple(0 for _ in shape))
    return pl.pallas_call(
        _node_body,
        grid=(N // NBLK,),
        in_specs=[
            pl.BlockSpec(memory_space=pltpu.SMEM),
            pl.BlockSpec((NBLK, D), lambda i: (i, 0)),
        ] + [
            pl.BlockSpec((2, NBLK, HID), lambda i: (0, i, 0))
            for _ in range(_K)
        ] + [
            blk((2 * D, HID)),
            blk((HID, NQ)),
            blk((1, HID)),
            blk((2 * D + NQ, HID)),
            blk((1, HID)),
            blk((HID, D)),
            blk((1, D)),
        ],
        out_specs=pl.BlockSpec((NBLK, D), lambda i: (i, 0)),
        out_shape=jax.ShapeDtypeStruct((N, D), jnp.float32),
    )(params, h, *parts, qw1, qw2, qb1.reshape(1, HID), pw1,
      pb1.reshape(1, HID), pw2, pb2.reshape(1, D))


def kernel(h, edge_index, ew1, eb1, ew2, eb2, qw1, qb1, qw2, qb2,
           pw1, pb1, pw2, pb2, alpha, beta, gamma, delta, Lam):
    row = edge_index[0]
    col = edge_index[1]
    p, q = _pq(h, ew1)
    mijs = []
    parts = []
    for k in range(_K):
        r_k = _sc_gather_add(p, q, row, col, k)
        mij_k = _emlp(r_k, eb1, ew2, eb2)
        mijs.append(mij_k)
        parts.append(_sc_segsum(mij_k, row, k))
    mij = jnp.concatenate(mijs, axis=0)
    h_out = _node_stage(h, parts, qw1, qb1, qw2, qb2,
                        pw1, pb1, pw2, pb2, alpha, beta, gamma, delta, Lam)
    return (h_out, mij)


# speedup vs baseline: 1.0973x; 1.0973x over previous
import dataclasses
import functools

import jax
import jax.numpy as jnp
from jax import lax
from jax.experimental import pallas as pl
from jax.experimental.pallas import tpu as pltpu
from jax.experimental.pallas import tpu_sc as plsc

N = 10000
E = 320000
D = 128
HID = 128
NQ = 3
NORM = 100.0
MU = 0.5

NBLK = 2000
EBLK = 2000


def _silu(x):
    return x * jax.nn.sigmoid(x)


def _pq_body(h_ref, w_ref, p_ref, q_ref):
    h = h_ref[...]
    p_ref[...] = jnp.dot(h, w_ref[:D, :], preferred_element_type=jnp.float32)
    q_ref[...] = jnp.dot(h, w_ref[D:, :], preferred_element_type=jnp.float32)


def _pq(h, ew1):
    return pl.pallas_call(
        _pq_body,
        grid=(N // NBLK,),
        in_specs=[
            pl.BlockSpec((NBLK, D), lambda i: (i, 0)),
            pl.BlockSpec((2 * D, HID), lambda i: (0, 0)),
        ],
        out_specs=[
            pl.BlockSpec((NBLK, HID), lambda i: (i, 0)),
            pl.BlockSpec((NBLK, HID), lambda i: (i, 0)),
        ],
        out_shape=[
            jax.ShapeDtypeStruct((N, HID), jnp.float32),
            jax.ShapeDtypeStruct((N, HID), jnp.float32),
        ],
    )(h, ew1)


_NW = 32
_EW = E // _NW
_G = 80
_NCH = _EW // _G

def _vmesh():
    return plsc.VectorSubcoreMesh(core_axis_name="c", subcore_axis_name="s")


def _sc_compiler_params():
    cp = pltpu.CompilerParams()
    if "needs_layout_passes" in pltpu.CompilerParams.__dataclass_fields__:
        cp = dataclasses.replace(cp, needs_layout_passes=False)
    return cp


_GNB = 3


def _sc_gather_body(p_hbm, q_hbm, row_hbm, col_hbm, r_hbm,
                    ri_v, ci_v, bp_v, bq_v, bo_v, gsem, ssem):
    wid = lax.axis_index("c") * 16 + lax.axis_index("s")
    base = wid * _EW
    pltpu.sync_copy(row_hbm.at[pl.ds(base, _EW)], ri_v)
    pltpu.sync_copy(col_hbm.at[pl.ds(base, _EW)], ci_v)

    def start_gather(c, slot):
        pltpu.async_copy(p_hbm.at[ri_v.at[pl.ds(c * _G, _G)]],
                         bp_v.at[slot], gsem.at[slot])
        pltpu.async_copy(q_hbm.at[ci_v.at[pl.ds(c * _G, _G)]],
                         bq_v.at[slot], gsem.at[slot])

    def wait_gather(slot):
        pltpu.make_async_copy(p_hbm.at[ri_v.at[pl.ds(0, _G)]],
                              bp_v.at[slot], gsem.at[slot]).wait()
        pltpu.make_async_copy(q_hbm.at[ci_v.at[pl.ds(0, _G)]],
                              bq_v.at[slot], gsem.at[slot]).wait()

    def wait_store(slot):
        pltpu.make_async_copy(bo_v.at[slot], r_hbm.at[pl.ds(base, _G)],
                              ssem.at[slot]).wait()

    def add_store(c, slot):
        bp = bp_v.at[slot]
        bq = bq_v.at[slot]
        bo = bo_v.at[slot]

        @pl.loop(0, _G)
        def _(r):
            for c0 in range(0, HID, 16):
                sl = (r, pl.ds(c0, 16))
                bo[sl] = bp[sl] + bq[sl]

        pltpu.async_copy(bo, r_hbm.at[pl.ds(base + c * _G, _G)], ssem.at[slot])

    for slot in range(_GNB):
        start_gather(slot, slot)
    _GMAIN = (_NCH // _GNB) * _GNB

    @pl.loop(0, _GMAIN, step=_GNB)
    def _(i):
        for slot in range(_GNB):
            c = i + slot
            wait_gather(slot)

            @pl.when(c >= _GNB)
            def _():
                wait_store(slot)

            add_store(c, slot)
            nxt = jnp.minimum(c + _GNB, _NCH - 1)
            start_gather(nxt, slot)

    for c in range(_GMAIN, _NCH):
        slot = c % _GNB
        wait_gather(slot)
        wait_store(slot)
        add_store(c, slot)
    wait_gather(2)
    for slot in range(_GNB):
        wait_store(slot)


def _sc_gather_add(p, q, row, col):
    f = pl.kernel(
        _sc_gather_body,
        out_type=jax.ShapeDtypeStruct((E, HID), jnp.float32),
        mesh=_vmesh(),
        scratch_types=[
            pltpu.VMEM((_EW,), jnp.int32),
            pltpu.VMEM((_EW,), jnp.int32),
            pltpu.VMEM((_GNB, _G, HID), jnp.float32),
            pltpu.VMEM((_GNB, _G, HID), jnp.float32),
            pltpu.VMEM((_GNB, _G, HID), jnp.float32),
            pltpu.SemaphoreType.DMA((_GNB,)),
            pltpu.SemaphoreType.DMA((_GNB,)),
        ],
        name="sc_gather",
        compiler_params=_sc_compiler_params(),
    )
    return f(p, q, row, col)


_TROWS = 624
_ZB = 48
_TAIL0 = 16 * _TROWS
_TAILN = N - _TAIL0

_SS = 8
_LOOK = 4
_GS = 40
_PASSES = 5
_PE = _EW // _PASSES
_PCH = _PE // _GS
_PMAIN = (_PCH // _SS) * _SS


def _sc_segsum_body(mij_hbm, row_hbm, out_hbm, ri_v, mb_v, zb_v, acc_sh,
                    lsem, asem):
    cid = lax.axis_index("c")
    sid = lax.axis_index("s")
    base = (cid * 16 + sid) * _EW

    @pl.loop(0, _ZB)
    def _(r):
        for c0 in range(0, HID, 16):
            zb_v[r, pl.ds(c0, 16)] = jnp.zeros((16,), jnp.float32)

    @pl.loop(0, _TROWS // _ZB)
    def _(i):
        pltpu.sync_copy(zb_v, acc_sh.at[pl.ds(sid * _TROWS + i * _ZB, _ZB)])

    @pl.when(sid == 0)
    def _():
        pltpu.sync_copy(zb_v.at[pl.ds(0, _TAILN)],
                        acc_sh.at[pl.ds(_TAIL0, _TAILN)])

    plsc.subcore_barrier()

    @pl.loop(0, _PASSES)
    def _(p):
        pbase = base + p * _PE
        pltpu.sync_copy(row_hbm.at[pl.ds(pbase, _PE)], ri_v)

        def start_load(c, slot):
            pltpu.async_copy(mij_hbm.at[pl.ds(pbase + c * _GS, _GS)],
                             mb_v.at[slot], lsem.at[slot])

        def wait_load(slot):
            pltpu.make_async_copy(mij_hbm.at[pl.ds(base, _GS)],
                                  mb_v.at[slot], lsem.at[slot]).wait()

        def start_add(c, slot):
            pltpu.async_copy(mb_v.at[slot],
                             acc_sh.at[ri_v.at[pl.ds(c * _GS, _GS)]],
                             asem.at[slot], add=True)

        def wait_add(slot):
            pltpu.make_async_copy(mb_v.at[slot],
                                  acc_sh.at[ri_v.at[pl.ds(0, _GS)]],
                                  asem.at[slot]).wait()

        for c in range(_LOOK):
            start_load(c, c)

        @pl.loop(0, _PMAIN, step=_SS)
        def _(i):
            for b in range(_SS):
                c = i + b
                nslot = (b + _LOOK) % _SS
                wait_load(b)
                start_add(c, b)

                @pl.when(c >= _LOOK)
                def _():
                    wait_add(nslot)

                @pl.when(c + _LOOK < _PCH)
                def _():
                    start_load(c + _LOOK, nslot)

        for c in range(_PMAIN, _PCH):
            wait_load(c % _SS)
            start_add(c, c % _SS)
        for c in range(_PMAIN - _LOOK, _PCH):
            wait_add(c % _SS)

    plsc.subcore_barrier()

    @pl.loop(0, _TROWS // _ZB)
    def _(i):
        r0 = sid * _TROWS + i * _ZB
        pltpu.sync_copy(acc_sh.at[pl.ds(r0, _ZB)],
                        out_hbm.at[cid, pl.ds(r0, _ZB)])

    @pl.when(sid == 0)
    def _():
        pltpu.sync_copy(acc_sh.at[pl.ds(_TAIL0, _TAILN)],
                        out_hbm.at[cid, pl.ds(_TAIL0, _TAILN)])


def _sc_segsum(mij, row):
    f = pl.kernel(
        _sc_segsum_body,
        out_type=jax.ShapeDtypeStruct((2, N, HID), jnp.float32),
        mesh=_vmesh(),
        scratch_types=[
            pltpu.VMEM((_PE,), jnp.int32),
            pltpu.VMEM((_SS, _GS, HID), jnp.float32),
            pltpu.VMEM((_ZB, HID), jnp.float32),
            pltpu.VMEM_SHARED((N, HID), jnp.float32),
            pltpu.SemaphoreType.DMA((_SS,)),
            pltpu.SemaphoreType.DMA((_SS,)),
        ],
        name="sc_segsum",
    )
    return f(mij, row)


def _emlp_body(r_ref, b1_ref, w2_ref, b2_ref, m_ref):
    hid = _silu(r_ref[...] + b1_ref[...])
    m_ref[...] = _silu(
        jnp.dot(hid, w2_ref[...], preferred_element_type=jnp.float32)
        + b2_ref[...]
    )


def _emlp(r, eb1, ew2, eb2):
    return pl.pallas_call(
        _emlp_body,
        grid=(E // EBLK,),
        in_specs=[
            pl.BlockSpec((EBLK, HID), lambda i: (i, 0)),
            pl.BlockSpec((1, HID), lambda i: (0, 0)),
            pl.BlockSpec((HID, HID), lambda i: (0, 0)),
            pl.BlockSpec((1, HID), lambda i: (0, 0)),
        ],
        out_specs=pl.BlockSpec((EBLK, HID), lambda i: (i, 0)),
        out_shape=jax.ShapeDtypeStruct((E, HID), jnp.float32),
    )(r, eb1.reshape(1, HID), ew2, eb2.reshape(1, HID))


_P_ALPHA, _P_SB, _P_CB, _P_DELTA, _P_BETA = 0, 1, 2, 3, 4
_P_QB2 = 5
_P_PHI = 8


def _node_body(params_ref, h_ref, pa_ref, qw1_ref, qw2_ref,
               qb1_ref, pw1_ref, pb1_ref, pw2_ref, pb2_ref, out_ref):
    h = h_ref[...]
    agg = (pa_ref[0] + pa_ref[1]) * (1.0 / NORM)
    cat = jnp.concatenate([h, agg], axis=1)
    hq = _silu(jnp.dot(cat, qw1_ref[...], preferred_element_type=jnp.float32)
               + qb1_ref[...])
    qin_t = lax.dot_general(qw2_ref[...], hq,
                            (((0,), (1,)), ((), ())),
                            preferred_element_type=jnp.float32)

    alpha = params_ref[_P_ALPHA]
    sb = params_ref[_P_SB]
    cb = params_ref[_P_CB]
    delta = params_ref[_P_DELTA]
    beta = params_ref[_P_BETA]

    q = [qin_t[k:k + 1, :] + params_ref[_P_QB2 + k] for k in range(NQ)]
    sa = [jnp.sin(alpha * qk) for qk in q]
    ca = [jnp.cos(alpha * qk) for qk in q]
    _pairidx = {(0, 1): 0, (0, 2): 1, (1, 2): 2}

    zrows = []
    for k in range(NQ):
        fr, fi = None, None
        for j in range(NQ):
            if j == k:
                continue
            pi = _pairidx[(min(k, j), max(k, j))]
            cp = params_ref[_P_PHI + 2 * pi]
            sp = params_ref[_P_PHI + 2 * pi + 1]
            gr = cp
            gi = -sp * ca[j]
            if fr is None:
                fr, fi = jnp.full_like(ca[j], gr), gi
            else:
                fr, fi = fr * gr - fi * gi, fr * gi + fi * gr
        qk = q[k]
        q2 = qk * qk
        d1 = delta * (1.0 - MU * q2)
        c = beta + delta * q2
        sc, cc = jnp.sin(c), jnp.cos(c)
        sd1, cd1 = jnp.sin(d1), jnp.cos(d1)
        rx = sa[k] * fr
        ry = -sa[k] * fi
        rz = ca[k]
        mx = sc * sd1
        my = sc * cd1 * cb + cc * sb
        mz = -sc * cd1 * sb + cc * cb
        zrows.append(rx * mx + ry * my + rz * mz)
    qout_t = jnp.concatenate(zrows, axis=0)

    hp_pre = (jnp.dot(cat, pw1_ref[:2 * D, :], preferred_element_type=jnp.float32)
              + lax.dot_general(qout_t, pw1_ref[2 * D:2 * D + NQ, :],
                                (((0,), (0,)), ((), ())),
                                preferred_element_type=jnp.float32)
              + pb1_ref[...])
    hp = _silu(hp_pre)
    out_ref[...] = h + jnp.dot(hp, pw2_ref[...],
                               preferred_element_type=jnp.float32) + pb2_ref[...]


def _node_stage(h, parts, qw1, qb1, qw2, qb2, pw1, pb1, pw2, pb2,
                alpha, beta, gamma, delta, Lam):
    phi = gamma * (Lam + Lam.T) / 2.0
    params = jnp.concatenate([
        jnp.stack([alpha, jnp.sin(beta), jnp.cos(beta), delta, beta]),
        qb2,
        jnp.stack([jnp.cos(phi[0, 1]), jnp.sin(phi[0, 1]),
                   jnp.cos(phi[0, 2]), jnp.sin(phi[0, 2]),
                   jnp.cos(phi[1, 2]), jnp.sin(phi[1, 2])]),
    ]).astype(jnp.float32)
    blk = lambda shape: pl.BlockSpec(shape, lambda i: tuple(0 for _ in shape))
    return pl.pallas_call(
        _node_body,
        grid=(N // NBLK,),
        in_specs=[
            pl.BlockSpec(memory_space=pltpu.SMEM),
            pl.BlockSpec((NBLK, D), lambda i: (i, 0)),
            pl.BlockSpec((2, NBLK, HID), lambda i: (0, i, 0)),
            blk((2 * D, HID)),
            blk((HID, NQ)),
            blk((1, HID)),
            blk((2 * D + NQ, HID)),
            blk((1, HID)),
            blk((HID, D)),
            blk((1, D)),
        ],
        out_specs=pl.BlockSpec((NBLK, D), lambda i: (i, 0)),
        out_shape=jax.ShapeDtypeStruct((N, D), jnp.float32),
    )(params, h, parts, qw1, qw2, qb1.reshape(1, HID), pw1,
      pb1.reshape(1, HID), pw2, pb2.reshape(1, D))


def kernel(h, edge_index, ew1, eb1, ew2, eb2, qw1, qb1, qw2, qb2,
           pw1, pb1, pw2, pb2, alpha, beta, gamma, delta, Lam):
    row = edge_index[0]
    col = edge_index[1]
    p, q = _pq(h, ew1)
    r = _sc_gather_add(p, q, row, col)
    mij = _emlp(r, eb1, ew2, eb2)
    parts = _sc_segsum(mij, row)
    h_out = _node_stage(h, parts, qw1, qb1, qw2, qb2,
                        pw1, pb1, pw2, pb2, alpha, beta, gamma, delta, Lam)
    return (h_out, mij)


# speedup vs baseline: 1.1011x; 1.0035x over previous
import dataclasses
import functools

import jax
import jax.numpy as jnp
from jax import lax
from jax.experimental import pallas as pl
from jax.experimental.pallas import tpu as pltpu
from jax.experimental.pallas import tpu_sc as plsc

N = 10000
E = 320000
D = 128
HID = 128
NQ = 3
NORM = 100.0
MU = 0.5

NBLK = 2000
EBLK = 2000


def _silu(x):
    return x * jax.nn.sigmoid(x)


def _pq_body(h_ref, w_ref, p_ref, q_ref):
    h = h_ref[...]
    p_ref[...] = jnp.dot(h, w_ref[:D, :], preferred_element_type=jnp.float32)
    q_ref[...] = jnp.dot(h, w_ref[D:, :], preferred_element_type=jnp.float32)


def _pq(h, ew1):
    return pl.pallas_call(
        _pq_body,
        grid=(N // NBLK,),
        in_specs=[
            pl.BlockSpec((NBLK, D), lambda i: (i, 0)),
            pl.BlockSpec((2 * D, HID), lambda i: (0, 0)),
        ],
        out_specs=[
            pl.BlockSpec((NBLK, HID), lambda i: (i, 0)),
            pl.BlockSpec((NBLK, HID), lambda i: (i, 0)),
        ],
        out_shape=[
            jax.ShapeDtypeStruct((N, HID), jnp.float32),
            jax.ShapeDtypeStruct((N, HID), jnp.float32),
        ],
    )(h, ew1)


_NW = 32
_K = 2
_EC = E // _K
_EW = _EC // _NW
_G = 40
_NCH = _EW // _G

def _vmesh():
    return plsc.VectorSubcoreMesh(core_axis_name="c", subcore_axis_name="s")


def _sc_compiler_params():
    cp = pltpu.CompilerParams()
    if "needs_layout_passes" in pltpu.CompilerParams.__dataclass_fields__:
        cp = dataclasses.replace(cp, needs_layout_passes=False)
    return cp


_GNB = 3


def _sc_gather_body(ofs, p_hbm, q_hbm, row_hbm, col_hbm, r_hbm,
                    ri_v, ci_v, bp_v, bq_v, bo_v, gsem, ssem):
    wid = lax.axis_index("c") * 16 + lax.axis_index("s")
    base = wid * _EW
    pltpu.sync_copy(row_hbm.at[pl.ds(ofs + base, _EW)], ri_v)
    pltpu.sync_copy(col_hbm.at[pl.ds(ofs + base, _EW)], ci_v)

    def start_gather(c, slot):
        pltpu.async_copy(p_hbm.at[ri_v.at[pl.ds(c * _G, _G)]],
                         bp_v.at[slot], gsem.at[slot])
        pltpu.async_copy(q_hbm.at[ci_v.at[pl.ds(c * _G, _G)]],
                         bq_v.at[slot], gsem.at[slot])

    def wait_gather(slot):
        pltpu.make_async_copy(p_hbm.at[ri_v.at[pl.ds(0, _G)]],
                              bp_v.at[slot], gsem.at[slot]).wait()
        pltpu.make_async_copy(q_hbm.at[ci_v.at[pl.ds(0, _G)]],
                              bq_v.at[slot], gsem.at[slot]).wait()

    def wait_store(slot):
        pltpu.make_async_copy(bo_v.at[slot], r_hbm.at[pl.ds(base, _G)],
                              ssem.at[slot]).wait()

    def add_store(c, slot):
        bp = bp_v.at[slot]
        bq = bq_v.at[slot]
        bo = bo_v.at[slot]

        @pl.loop(0, _G)
        def _(r):
            for c0 in range(0, HID, 16):
                sl = (r, pl.ds(c0, 16))
                bo[sl] = bp[sl] + bq[sl]

        pltpu.async_copy(bo, r_hbm.at[pl.ds(base + c * _G, _G)], ssem.at[slot])

    for slot in range(_GNB):
        start_gather(slot, slot)
    _GMAIN = (_NCH // _GNB) * _GNB

    @pl.loop(0, _GMAIN, step=_GNB)
    def _(i):
        for slot in range(_GNB):
            c = i + slot
            wait_gather(slot)

            @pl.when(c >= _GNB)
            def _():
                wait_store(slot)

            add_store(c, slot)
            nxt = jnp.minimum(c + _GNB, _NCH - 1)
            start_gather(nxt, slot)

    for c in range(_GMAIN, _NCH):
        slot = c % _GNB
        wait_gather(slot)
        wait_store(slot)
        add_store(c, slot)
    wait_gather(2)
    for slot in range(_GNB):
        wait_store(slot)


def _sc_gather_add(p, q, row, col, k):
    f = pl.kernel(
        functools.partial(_sc_gather_body, k * _EC),
        out_type=jax.ShapeDtypeStruct((_EC, HID), jnp.float32),
        mesh=_vmesh(),
        scratch_types=[
            pltpu.VMEM((_EW,), jnp.int32),
            pltpu.VMEM((_EW,), jnp.int32),
            pltpu.VMEM((_GNB, _G, HID), jnp.float32),
            pltpu.VMEM((_GNB, _G, HID), jnp.float32),
            pltpu.VMEM((_GNB, _G, HID), jnp.float32),
            pltpu.SemaphoreType.DMA((_GNB,)),
            pltpu.SemaphoreType.DMA((_GNB,)),
        ],
        name=f"sc_gather_{k}",
        compiler_params=_sc_compiler_params(),
    )
    return f(p, q, row, col)


_TROWS = 624
_ZB = 48
_TAIL0 = 16 * _TROWS
_TAILN = N - _TAIL0

_SS = 8
_LOOK = 4
_GS = 40
_PASSES = 5
_PE = _EW // _PASSES
_PCH = _PE // _GS
_PMAIN = (_PCH // _SS) * _SS


def _sc_segsum_body(ofs, mij_hbm, row_hbm, out_hbm, ri_v, mb_v, zb_v, acc_sh,
                    lsem, asem):
    cid = lax.axis_index("c")
    sid = lax.axis_index("s")
    base = (cid * 16 + sid) * _EW

    @pl.loop(0, _ZB)
    def _(r):
        for c0 in range(0, HID, 16):
            zb_v[r, pl.ds(c0, 16)] = jnp.zeros((16,), jnp.float32)

    @pl.loop(0, _TROWS // _ZB)
    def _(i):
        pltpu.sync_copy(zb_v, acc_sh.at[pl.ds(sid * _TROWS + i * _ZB, _ZB)])

    @pl.when(sid == 0)
    def _():
        pltpu.sync_copy(zb_v.at[pl.ds(0, _TAILN)],
                        acc_sh.at[pl.ds(_TAIL0, _TAILN)])

    plsc.subcore_barrier()

    @pl.loop(0, _PASSES)
    def _(p):
        pbase = base + p * _PE
        pltpu.sync_copy(row_hbm.at[pl.ds(ofs + pbase, _PE)], ri_v)

        def start_load(c, slot):
            pltpu.async_copy(mij_hbm.at[pl.ds(pbase + c * _GS, _GS)],
                             mb_v.at[slot], lsem.at[slot])

        def wait_load(slot):
            pltpu.make_async_copy(mij_hbm.at[pl.ds(base, _GS)],
                                  mb_v.at[slot], lsem.at[slot]).wait()

        def start_add(c, slot):
            pltpu.async_copy(mb_v.at[slot],
                             acc_sh.at[ri_v.at[pl.ds(c * _GS, _GS)]],
                             asem.at[slot], add=True)

        def wait_add(slot):
            pltpu.make_async_copy(mb_v.at[slot],
                                  acc_sh.at[ri_v.at[pl.ds(0, _GS)]],
                                  asem.at[slot]).wait()

        for c in range(_LOOK):
            start_load(c, c)

        @pl.loop(0, _PMAIN, step=_SS)
        def _(i):
            for b in range(_SS):
                c = i + b
                nslot = (b + _LOOK) % _SS
                wait_load(b)
                start_add(c, b)

                @pl.when(c >= _LOOK)
                def _():
                    wait_add(nslot)

                @pl.when(c + _LOOK < _PCH)
                def _():
                    start_load(c + _LOOK, nslot)

        for c in range(_PMAIN, _PCH):
            wait_load(c % _SS)
            start_add(c, c % _SS)
        for c in range(_PMAIN - _LOOK, _PCH):
            wait_add(c % _SS)

    plsc.subcore_barrier()

    @pl.loop(0, _TROWS // _ZB)
    def _(i):
        r0 = sid * _TROWS + i * _ZB
        pltpu.sync_copy(acc_sh.at[pl.ds(r0, _ZB)],
                        out_hbm.at[cid, pl.ds(r0, _ZB)])

    @pl.when(sid == 0)
    def _():
        pltpu.sync_copy(acc_sh.at[pl.ds(_TAIL0, _TAILN)],
                        out_hbm.at[cid, pl.ds(_TAIL0, _TAILN)])


def _sc_segsum(mij, row, k):
    f = pl.kernel(
        functools.partial(_sc_segsum_body, k * _EC),
        out_type=jax.ShapeDtypeStruct((2, N, HID), jnp.float32),
        mesh=_vmesh(),
        scratch_types=[
            pltpu.VMEM((_PE,), jnp.int32),
            pltpu.VMEM((_SS, _GS, HID), jnp.float32),
            pltpu.VMEM((_ZB, HID), jnp.float32),
            pltpu.VMEM_SHARED((N, HID), jnp.float32),
            pltpu.SemaphoreType.DMA((_SS,)),
            pltpu.SemaphoreType.DMA((_SS,)),
        ],
        name=f"sc_segsum_{k}",
    )
    return f(mij, row)


def _emlp_body(r_ref, b1_ref, w2_ref, b2_ref, m_ref):
    hid = _silu(r_ref[...] + b1_ref[...])
    m_ref[...] = _silu(
        jnp.dot(hid, w2_ref[...], preferred_element_type=jnp.float32)
        + b2_ref[...]
    )


def _emlp(r, eb1, ew2, eb2):
    return pl.pallas_call(
        _emlp_body,
        grid=(_EC // EBLK,),
        in_specs=[
            pl.BlockSpec((EBLK, HID), lambda i: (i, 0)),
            pl.BlockSpec((1, HID), lambda i: (0, 0)),
            pl.BlockSpec((HID, HID), lambda i: (0, 0)),
            pl.BlockSpec((1, HID), lambda i: (0, 0)),
        ],
        out_specs=pl.BlockSpec((EBLK, HID), lambda i: (i, 0)),
        out_shape=jax.ShapeDtypeStruct((_EC, HID), jnp.float32),
    )(r, eb1.reshape(1, HID), ew2, eb2.reshape(1, HID))


_P_ALPHA, _P_SB, _P_CB, _P_DELTA, _P_BETA = 0, 1, 2, 3, 4
_P_QB2 = 5
_P_PHI = 8


def _node_body(params_ref, h_ref, *rest):
    (pa_refs, (qw1_ref, qw2_ref, qb1_ref, pw1_ref, pb1_ref, pw2_ref,
               pb2_ref, out_ref)) = rest[:_K], rest[_K:]
    h = h_ref[...]
    acc = pa_refs[0][0] + pa_refs[0][1]
    for _k in range(1, _K):
        acc = acc + pa_refs[_k][0] + pa_refs[_k][1]
    agg = acc * (1.0 / NORM)
    cat = jnp.concatenate([h, agg], axis=1)
    hq = _silu(jnp.dot(cat, qw1_ref[...], preferred_element_type=jnp.float32)
               + qb1_ref[...])
    qin_t = lax.dot_general(qw2_ref[...], hq,
                            (((0,), (1,)), ((), ())),
                            preferred_element_type=jnp.float32)

    alpha = params_ref[_P_ALPHA]
    sb = params_ref[_P_SB]
    cb = params_ref[_P_CB]
    delta = params_ref[_P_DELTA]
    beta = params_ref[_P_BETA]

    q = [qin_t[k:k + 1, :] + params_ref[_P_QB2 + k] for k in range(NQ)]
    sa = [jnp.sin(alpha * qk) for qk in q]
    ca = [jnp.cos(alpha * qk) for qk in q]
    _pairidx = {(0, 1): 0, (0, 2): 1, (1, 2): 2}

    zrows = []
    for k in range(NQ):
        fr, fi = None, None
        for j in range(NQ):
            if j == k:
                continue
            pi = _pairidx[(min(k, j), max(k, j))]
            cp = params_ref[_P_PHI + 2 * pi]
            sp = params_ref[_P_PHI + 2 * pi + 1]
            gr = cp
            gi = -sp * ca[j]
            if fr is None:
                fr, fi = jnp.full_like(ca[j], gr), gi
            else:
                fr, fi = fr * gr - fi * gi, fr * gi + fi * gr
        qk = q[k]
        q2 = qk * qk
        d1 = delta * (1.0 - MU * q2)
        c = beta + delta * q2
        sc, cc = jnp.sin(c), jnp.cos(c)
        sd1, cd1 = jnp.sin(d1), jnp.cos(d1)
        rx = sa[k] * fr
        ry = -sa[k] * fi
        rz = ca[k]
        mx = sc * sd1
        my = sc * cd1 * cb + cc * sb
        mz = -sc * cd1 * sb + cc * cb
        zrows.append(rx * mx + ry * my + rz * mz)
    qout_t = jnp.concatenate(zrows, axis=0)

    hp_pre = (jnp.dot(cat, pw1_ref[:2 * D, :], preferred_element_type=jnp.float32)
              + lax.dot_general(qout_t, pw1_ref[2 * D:2 * D + NQ, :],
                                (((0,), (0,)), ((), ())),
                                preferred_element_type=jnp.float32)
              + pb1_ref[...])
    hp = _silu(hp_pre)
    out_ref[...] = h + jnp.dot(hp, pw2_ref[...],
                               preferred_element_type=jnp.float32) + pb2_ref[...]


def _node_stage(h, parts, qw1, qb1, qw2, qb2, pw1, pb1, pw2, pb2,
                alpha, beta, gamma, delta, Lam):
    phi = gamma * (Lam + Lam.T) / 2.0
    params = jnp.concatenate([
        jnp.stack([alpha, jnp.sin(beta), jnp.cos(beta), delta, beta]),
        qb2,
        jnp.stack([jnp.cos(phi[0, 1]), jnp.sin(phi[0, 1]),
                   jnp.cos(phi[0, 2]), jnp.sin(phi[0, 2]),
                   jnp.cos(phi[1, 2]), jnp.sin(phi[1, 2])]),
    ]).astype(jnp.float32)
    blk = lambda shape: pl.BlockSpec(shape, lambda i: tuple(0 for _ in shape))
    return pl.pallas_call(
        _node_body,
        grid=(N // NBLK,),
        in_specs=[
            pl.BlockSpec(memory_space=pltpu.SMEM),
            pl.BlockSpec((NBLK, D), lambda i: (i, 0)),
        ] + [
            pl.BlockSpec((2, NBLK, HID), lambda i: (0, i, 0))
            for _ in range(_K)
        ] + [
            blk((2 * D, HID)),
            blk((HID, NQ)),
            blk((1, HID)),
            blk((2 * D + NQ, HID)),
            blk((1, HID)),
            blk((HID, D)),
            blk((1, D)),
        ],
        out_specs=pl.BlockSpec((NBLK, D), lambda i: (i, 0)),
        out_shape=jax.ShapeDtypeStruct((N, D), jnp.float32),
    )(params, h, *parts, qw1, qw2, qb1.reshape(1, HID), pw1,
      pb1.reshape(1, HID), pw2, pb2.reshape(1, D))


def kernel(h, edge_index, ew1, eb1, ew2, eb2, qw1, qb1, qw2, qb2,
           pw1, pb1, pw2, pb2, alpha, beta, gamma, delta, Lam):
    row = edge_index[0]
    col = edge_index[1]
    p, q = _pq(h, ew1)
    mijs = []
    parts = []
    for k in range(_K):
        r_k = _sc_gather_add(p, q, row, col, k)
        mij_k = _emlp(r_k, eb1, ew2, eb2)
        mijs.append(mij_k)
        parts.append(_sc_segsum(mij_k, row, k))
    mij = jnp.concatenate(mijs, axis=0)
    h_out = _node_stage(h, parts, qw1, qb1, qw2, qb2,
                        pw1, pb1, pw2, pb2, alpha, beta, gamma, delta, Lam)
    return (h_out, mij)


# speedup vs baseline: 1.1536x; 1.0477x over previous
import dataclasses
import functools

import jax
import jax.numpy as jnp
from jax import lax
from jax.experimental import pallas as pl
from jax.experimental.pallas import tpu as pltpu
from jax.experimental.pallas import tpu_sc as plsc

N = 10000
E = 320000
D = 128
HID = 128
NQ = 3
NORM = 100.0
MU = 0.5

NBLK = 2000
EBLK = 2000


def _silu(x):
    return x * jax.nn.sigmoid(x)


def _pq_body(h_ref, w_ref, p_ref, q_ref):
    h = h_ref[...]
    p_ref[...] = jnp.dot(h, w_ref[:D, :], preferred_element_type=jnp.float32)
    q_ref[...] = jnp.dot(h, w_ref[D:, :], preferred_element_type=jnp.float32)


def _pq(h, ew1):
    return pl.pallas_call(
        _pq_body,
        grid=(N // NBLK,),
        in_specs=[
            pl.BlockSpec((NBLK, D), lambda i: (i, 0)),
            pl.BlockSpec((2 * D, HID), lambda i: (0, 0)),
        ],
        out_specs=[
            pl.BlockSpec((NBLK, HID), lambda i: (i, 0)),
            pl.BlockSpec((NBLK, HID), lambda i: (i, 0)),
        ],
        out_shape=[
            jax.ShapeDtypeStruct((N, HID), jnp.float32),
            jax.ShapeDtypeStruct((N, HID), jnp.float32),
        ],
    )(h, ew1)


_NW = 32
_K = 2
_EC = E // _K
_EW = _EC // _NW
_G = 40
_NCH = _EW // _G

def _vmesh():
    return plsc.VectorSubcoreMesh(core_axis_name="c", subcore_axis_name="s")


def _sc_compiler_params():
    cp = pltpu.CompilerParams()
    if "needs_layout_passes" in pltpu.CompilerParams.__dataclass_fields__:
        cp = dataclasses.replace(cp, needs_layout_passes=False)
    return cp


_GNB = 3


def _sc_gather_body(ofs, p_hbm, q_hbm, row_hbm, col_hbm, r_hbm,
                    ri_v, ci_v, bp_v, bq_v, bo_v, gsem, ssem):
    wid = lax.axis_index("c") * 16 + lax.axis_index("s")
    base = wid * _EW
    pltpu.sync_copy(row_hbm.at[pl.ds(ofs + base, _EW)], ri_v)
    pltpu.sync_copy(col_hbm.at[pl.ds(ofs + base, _EW)], ci_v)

    def start_gather(c, slot):
        pltpu.async_copy(p_hbm.at[ri_v.at[pl.ds(c * _G, _G)]],
                         bp_v.at[slot], gsem.at[slot])
        pltpu.async_copy(q_hbm.at[ci_v.at[pl.ds(c * _G, _G)]],
                         bq_v.at[slot], gsem.at[slot])

    def wait_gather(slot):
        pltpu.make_async_copy(p_hbm.at[ri_v.at[pl.ds(0, _G)]],
                              bp_v.at[slot], gsem.at[slot]).wait()
        pltpu.make_async_copy(q_hbm.at[ci_v.at[pl.ds(0, _G)]],
                              bq_v.at[slot], gsem.at[slot]).wait()

    def wait_store(slot):
        pltpu.make_async_copy(bo_v.at[slot], r_hbm.at[pl.ds(base, _G)],
                              ssem.at[slot]).wait()

    def add_store(c, slot):
        bp = bp_v.at[slot]
        bq = bq_v.at[slot]
        bo = bo_v.at[slot]

        @pl.loop(0, _G)
        def _(r):
            for c0 in range(0, HID, 16):
                sl = (r, pl.ds(c0, 16))
                bo[sl] = bp[sl] + bq[sl]

        pltpu.async_copy(bo, r_hbm.at[pl.ds(base + c * _G, _G)], ssem.at[slot])

    for slot in range(_GNB):
        start_gather(slot, slot)
    _GMAIN = (_NCH // _GNB) * _GNB

    @pl.loop(0, _GMAIN, step=_GNB)
    def _(i):
        for slot in range(_GNB):
            c = i + slot
            wait_gather(slot)

            @pl.when(c >= _GNB)
            def _():
                wait_store(slot)

            add_store(c, slot)
            nxt = jnp.minimum(c + _GNB, _NCH - 1)
            start_gather(nxt, slot)

    for c in range(_GMAIN, _NCH):
        slot = c % _GNB
        wait_gather(slot)
        wait_store(slot)
        add_store(c, slot)
    wait_gather(2)
    for slot in range(_GNB):
        wait_store(slot)


def _sc_gather_add(p, q, row, col, k):
    f = pl.kernel(
        functools.partial(_sc_gather_body, k * _EC),
        out_type=jax.ShapeDtypeStruct((_EC, HID), jnp.float32),
        mesh=_vmesh(),
        scratch_types=[
            pltpu.VMEM((_EW,), jnp.int32),
            pltpu.VMEM((_EW,), jnp.int32),
            pltpu.VMEM((_GNB, _G, HID), jnp.float32),
            pltpu.VMEM((_GNB, _G, HID), jnp.float32),
            pltpu.VMEM((_GNB, _G, HID), jnp.float32),
            pltpu.SemaphoreType.DMA((_GNB,)),
            pltpu.SemaphoreType.DMA((_GNB,)),
        ],
        name=f"sc_gather_{k}",
        compiler_params=_sc_compiler_params(),
    )
    return f(p, q, row, col)


_TROWS = 624
_ZB = 48
_TAIL0 = 16 * _TROWS
_TAILN = N - _TAIL0

_SS = 8
_LOOK = 4
_GS = 40
_PASSES = 5
_PE = _EW // _PASSES
_PCH = _PE // _GS
_PMAIN = (_PCH // _SS) * _SS


def _sc_segsum_body(ofs, mij_hbm, row_hbm, out_hbm, ri_v, mb_v, zb_v, acc_sh,
                    lsem, asem):
    cid = lax.axis_index("c")
    sid = lax.axis_index("s")
    base = (cid * 16 + sid) * _EW

    @pl.loop(0, _ZB)
    def _(r):
        for c0 in range(0, HID, 16):
            zb_v[r, pl.ds(c0, 16)] = jnp.zeros((16,), jnp.float32)

    @pl.loop(0, _TROWS // _ZB)
    def _(i):
        pltpu.sync_copy(zb_v, acc_sh.at[pl.ds(sid * _TROWS + i * _ZB, _ZB)])

    @pl.when(sid == 0)
    def _():
        pltpu.sync_copy(zb_v.at[pl.ds(0, _TAILN)],
                        acc_sh.at[pl.ds(_TAIL0, _TAILN)])

    plsc.subcore_barrier()

    @pl.loop(0, _PASSES)
    def _(p):
        pbase = base + p * _PE
        pltpu.sync_copy(row_hbm.at[pl.ds(ofs + pbase, _PE)], ri_v)

        def start_load(c, slot):
            pltpu.async_copy(mij_hbm.at[pl.ds(pbase + c * _GS, _GS)],
                             mb_v.at[slot], lsem.at[slot])

        def wait_load(slot):
            pltpu.make_async_copy(mij_hbm.at[pl.ds(base, _GS)],
                                  mb_v.at[slot], lsem.at[slot]).wait()

        def start_add(c, slot):
            pltpu.async_copy(mb_v.at[slot],
                             acc_sh.at[ri_v.at[pl.ds(c * _GS, _GS)]],
                             asem.at[slot], add=True)

        def wait_add(slot):
            pltpu.make_async_copy(mb_v.at[slot],
                                  acc_sh.at[ri_v.at[pl.ds(0, _GS)]],
                                  asem.at[slot]).wait()

        for c in range(_LOOK):
            start_load(c, c)

        @pl.loop(0, _PMAIN, step=_SS)
        def _(i):
            for b in range(_SS):
                c = i + b
                nslot = (b + _LOOK) % _SS
                wait_load(b)
                start_add(c, b)

                @pl.when(c >= _LOOK)
                def _():
                    wait_add(nslot)

                @pl.when(c + _LOOK < _PCH)
                def _():
                    start_load(c + _LOOK, nslot)

        for c in range(_PMAIN, _PCH):
            wait_load(c % _SS)
            start_add(c, c % _SS)
        for c in range(_PMAIN - _LOOK, _PCH):
            wait_add(c % _SS)

    plsc.subcore_barrier()

    @pl.loop(0, _TROWS // _ZB)
    def _(i):
        r0 = sid * _TROWS + i * _ZB
        pltpu.sync_copy(acc_sh.at[pl.ds(r0, _ZB)],
                        out_hbm.at[cid, pl.ds(r0, _ZB)])

    @pl.when(sid == 0)
    def _():
        pltpu.sync_copy(acc_sh.at[pl.ds(_TAIL0, _TAILN)],
                        out_hbm.at[cid, pl.ds(_TAIL0, _TAILN)])


def _sc_segsum(mij, row, k):
    f = pl.kernel(
        functools.partial(_sc_segsum_body, k * _EC),
        out_type=jax.ShapeDtypeStruct((2, N, HID), jnp.float32),
        mesh=_vmesh(),
        scratch_types=[
            pltpu.VMEM((_PE,), jnp.int32),
            pltpu.VMEM((_SS, _GS, HID), jnp.float32),
            pltpu.VMEM((_ZB, HID), jnp.float32),
            pltpu.VMEM_SHARED((N, HID), jnp.float32),
            pltpu.SemaphoreType.DMA((_SS,)),
            pltpu.SemaphoreType.DMA((_SS,)),
        ],
        name=f"sc_segsum_{k}",
    )
    return f(mij, row)


def _emlp_body0(r_ref, b1_ref, w2_ref, b2_ref, mfull_ref, m_ref):
    hid = _silu(r_ref[...] + b1_ref[...])
    v = _silu(
        jnp.dot(hid, w2_ref[...], preferred_element_type=jnp.float32)
        + b2_ref[...]
    )
    mfull_ref[...] = v
    m_ref[...] = v


def _emlp_body1(mprev_ref, r_ref, b1_ref, w2_ref, b2_ref, mfull_ref, m_ref):
    _emlp_body0(r_ref, b1_ref, w2_ref, b2_ref, mfull_ref, m_ref)


def _emlp(r, eb1, ew2, eb2, k, mij_prev):
    nblk = _EC // EBLK
    common_in = [
        pl.BlockSpec((EBLK, HID), lambda i: (i, 0)),
        pl.BlockSpec((1, HID), lambda i: (0, 0)),
        pl.BlockSpec((HID, HID), lambda i: (0, 0)),
        pl.BlockSpec((1, HID), lambda i: (0, 0)),
    ]
    out_specs = [
        pl.BlockSpec((EBLK, HID), lambda i, k=k, n=nblk: (i + k * n, 0)),
        pl.BlockSpec((EBLK, HID), lambda i: (i, 0)),
    ]
    out_shape = [
        jax.ShapeDtypeStruct((E, HID), jnp.float32),
        jax.ShapeDtypeStruct((_EC, HID), jnp.float32),
    ]
    args = (r, eb1.reshape(1, HID), ew2, eb2.reshape(1, HID))
    if k == 0:
        return pl.pallas_call(
            _emlp_body0,
            grid=(nblk,),
            in_specs=common_in,
            out_specs=out_specs,
            out_shape=out_shape,
        )(*args)
    return pl.pallas_call(
        _emlp_body1,
        grid=(nblk,),
        in_specs=[pl.BlockSpec(memory_space=pl.ANY)] + common_in,
        out_specs=out_specs,
        out_shape=out_shape,
        input_output_aliases={0: 0},
    )(mij_prev, *args)


_P_ALPHA, _P_SB, _P_CB, _P_DELTA, _P_BETA = 0, 1, 2, 3, 4
_P_QB2 = 5
_P_PHI = 8


def _node_body(params_ref, h_ref, *rest):
    (pa_refs, (qw1_ref, qw2_ref, qb1_ref, pw1_ref, pb1_ref, pw2_ref,
               pb2_ref, out_ref)) = rest[:_K], rest[_K:]
    h = h_ref[...]
    acc = pa_refs[0][0] + pa_refs[0][1]
    for _k in range(1, _K):
        acc = acc + pa_refs[_k][0] + pa_refs[_k][1]
    agg = acc * (1.0 / NORM)
    cat = jnp.concatenate([h, agg], axis=1)
    hq = _silu(jnp.dot(cat, qw1_ref[...], preferred_element_type=jnp.float32)
               + qb1_ref[...])
    qin_t = lax.dot_general(qw2_ref[...], hq,
                            (((0,), (1,)), ((), ())),
                            preferred_element_type=jnp.float32)

    alpha = params_ref[_P_ALPHA]
    sb = params_ref[_P_SB]
    cb = params_ref[_P_CB]
    delta = params_ref[_P_DELTA]
    beta = params_ref[_P_BETA]

    q = [qin_t[k:k + 1, :] + params_ref[_P_QB2 + k] for k in range(NQ)]
    sa = [jnp.sin(alpha * qk) for qk in q]
    ca = [jnp.cos(alpha * qk) for qk in q]
    _pairidx = {(0, 1): 0, (0, 2): 1, (1, 2): 2}

    zrows = []
    for k in range(NQ):
        fr, fi = None, None
        for j in range(NQ):
            if j == k:
                continue
            pi = _pairidx[(min(k, j), max(k, j))]
            cp = params_ref[_P_PHI + 2 * pi]
            sp = params_ref[_P_PHI + 2 * pi + 1]
            gr = cp
            gi = -sp * ca[j]
            if fr is None:
                fr, fi = jnp.full_like(ca[j], gr), gi
            else:
                fr, fi = fr * gr - fi * gi, fr * gi + fi * gr
        qk = q[k]
        q2 = qk * qk
        d1 = delta * (1.0 - MU * q2)
        c = beta + delta * q2
        sc, cc = jnp.sin(c), jnp.cos(c)
        sd1, cd1 = jnp.sin(d1), jnp.cos(d1)
        rx = sa[k] * fr
        ry = -sa[k] * fi
        rz = ca[k]
        mx = sc * sd1
        my = sc * cd1 * cb + cc * sb
        mz = -sc * cd1 * sb + cc * cb
        zrows.append(rx * mx + ry * my + rz * mz)
    qout_t = jnp.concatenate(zrows, axis=0)

    hp_pre = (jnp.dot(cat, pw1_ref[:2 * D, :], preferred_element_type=jnp.float32)
              + lax.dot_general(qout_t, pw1_ref[2 * D:2 * D + NQ, :],
                                (((0,), (0,)), ((), ())),
                                preferred_element_type=jnp.float32)
              + pb1_ref[...])
    hp = _silu(hp_pre)
    out_ref[...] = h + jnp.dot(hp, pw2_ref[...],
                               preferred_element_type=jnp.float32) + pb2_ref[...]


def _node_stage(h, parts, qw1, qb1, qw2, qb2, pw1, pb1, pw2, pb2,
                alpha, beta, gamma, delta, Lam):
    phi = gamma * (Lam + Lam.T) / 2.0
    params = jnp.concatenate([
        jnp.stack([alpha, jnp.sin(beta), jnp.cos(beta), delta, beta]),
        qb2,
        jnp.stack([jnp.cos(phi[0, 1]), jnp.sin(phi[0, 1]),
                   jnp.cos(phi[0, 2]), jnp.sin(phi[0, 2]),
                   jnp.cos(phi[1, 2]), jnp.sin(phi[1, 2])]),
    ]).astype(jnp.float32)
    blk = lambda shape: pl.BlockSpec(shape, lambda i: tuple(0 for _ in shape))
    return pl.pallas_call(
        _node_body,
        grid=(N // NBLK,),
        in_specs=[
            pl.BlockSpec(memory_space=pltpu.SMEM),
            pl.BlockSpec((NBLK, D), lambda i: (i, 0)),
        ] + [
            pl.BlockSpec((2, NBLK, HID), lambda i: (0, i, 0))
            for _ in range(_K)
        ] + [
            blk((2 * D, HID)),
            blk((HID, NQ)),
            blk((1, HID)),
            blk((2 * D + NQ, HID)),
            blk((1, HID)),
            blk((HID, D)),
            blk((1, D)),
        ],
        out_specs=pl.BlockSpec((NBLK, D), lambda i: (i, 0)),
        out_shape=jax.ShapeDtypeStruct((N, D), jnp.float32),
    )(params, h, *parts, qw1, qw2, qb1.reshape(1, HID), pw1,
      pb1.reshape(1, HID), pw2, pb2.reshape(1, D))


def kernel(h, edge_index, ew1, eb1, ew2, eb2, qw1, qb1, qw2, qb2,
           pw1, pb1, pw2, pb2, alpha, beta, gamma, delta, Lam):
    row = edge_index[0]
    col = edge_index[1]
    p, q = _pq(h, ew1)
    parts = []
    mij = None
    for k in range(_K):
        r_k = _sc_gather_add(p, q, row, col, k)
        mij, mij_k = _emlp(r_k, eb1, ew2, eb2, k, mij)
        parts.append(_sc_segsum(mij_k, row, k))
    h_out = _node_stage(h, parts, qw1, qb1, qw2, qb2,
                        pw1, pb1, pw2, pb2, alpha, beta, gamma, delta, Lam)
    return (h_out, mij)


# speedup vs baseline: 1.1738x; 1.0174x over previous
import dataclasses
import functools

import jax
import jax.numpy as jnp
from jax import lax
from jax.experimental import pallas as pl
from jax.experimental.pallas import tpu as pltpu
from jax.experimental.pallas import tpu_sc as plsc

N = 10000
E = 320000
D = 128
HID = 128
NQ = 3
NORM = 100.0
MU = 0.5

NBLK = 2000
EBLK = 4000


def _silu(x):
    return x * jax.nn.sigmoid(x)


def _pq_body(h_ref, w_ref, p_ref, q_ref):
    h = h_ref[...]
    p_ref[...] = jnp.dot(h, w_ref[:D, :], preferred_element_type=jnp.float32)
    q_ref[...] = jnp.dot(h, w_ref[D:, :], preferred_element_type=jnp.float32)


def _pq(h, ew1):
    return pl.pallas_call(
        _pq_body,
        grid=(N // NBLK,),
        in_specs=[
            pl.BlockSpec((NBLK, D), lambda i: (i, 0)),
            pl.BlockSpec((2 * D, HID), lambda i: (0, 0)),
        ],
        out_specs=[
            pl.BlockSpec((NBLK, HID), lambda i: (i, 0)),
            pl.BlockSpec((NBLK, HID), lambda i: (i, 0)),
        ],
        out_shape=[
            jax.ShapeDtypeStruct((N, HID), jnp.float32),
            jax.ShapeDtypeStruct((N, HID), jnp.float32),
        ],
    )(h, ew1)


_NW = 32
_K = 2
_EC = E // _K
_EW = _EC // _NW
_G = 40
_NCH = _EW // _G

def _vmesh():
    return plsc.VectorSubcoreMesh(core_axis_name="c", subcore_axis_name="s")


def _sc_compiler_params():
    cp = pltpu.CompilerParams()
    if "needs_layout_passes" in pltpu.CompilerParams.__dataclass_fields__:
        cp = dataclasses.replace(cp, needs_layout_passes=False)
    return cp


_GNB = 3


def _sc_gather_body(ofs, p_hbm, q_hbm, row_hbm, col_hbm, r_hbm,
                    ri_v, ci_v, bp_v, bq_v, bo_v, gsem, ssem):
    wid = lax.axis_index("c") * 16 + lax.axis_index("s")
    base = wid * _EW
    pltpu.sync_copy(row_hbm.at[pl.ds(ofs + base, _EW)], ri_v)
    pltpu.sync_copy(col_hbm.at[pl.ds(ofs + base, _EW)], ci_v)

    def start_gather(c, slot):
        pltpu.async_copy(p_hbm.at[ri_v.at[pl.ds(c * _G, _G)]],
                         bp_v.at[slot], gsem.at[slot])
        pltpu.async_copy(q_hbm.at[ci_v.at[pl.ds(c * _G, _G)]],
                         bq_v.at[slot], gsem.at[slot])

    def wait_gather(slot):
        pltpu.make_async_copy(p_hbm.at[ri_v.at[pl.ds(0, _G)]],
                              bp_v.at[slot], gsem.at[slot]).wait()
        pltpu.make_async_copy(q_hbm.at[ci_v.at[pl.ds(0, _G)]],
                              bq_v.at[slot], gsem.at[slot]).wait()

    def wait_store(slot):
        pltpu.make_async_copy(bo_v.at[slot], r_hbm.at[pl.ds(base, _G)],
                              ssem.at[slot]).wait()

    def add_store(c, slot):
        bp = bp_v.at[slot]
        bq = bq_v.at[slot]
        bo = bo_v.at[slot]

        @pl.loop(0, _G)
        def _(r):
            for c0 in range(0, HID, 16):
                sl = (r, pl.ds(c0, 16))
                bo[sl] = bp[sl] + bq[sl]

        pltpu.async_copy(bo, r_hbm.at[pl.ds(base + c * _G, _G)], ssem.at[slot])

    for slot in range(_GNB):
        start_gather(slot, slot)
    _GMAIN = (_NCH // _GNB) * _GNB

    @pl.loop(0, _GMAIN, step=_GNB)
    def _(i):
        for slot in range(_GNB):
            c = i + slot
            wait_gather(slot)

            @pl.when(c >= _GNB)
            def _():
                wait_store(slot)

            add_store(c, slot)
            nxt = jnp.minimum(c + _GNB, _NCH - 1)
            start_gather(nxt, slot)

    for c in range(_GMAIN, _NCH):
        slot = c % _GNB
        wait_gather(slot)
        wait_store(slot)
        add_store(c, slot)
    wait_gather(2)
    for slot in range(_GNB):
        wait_store(slot)


def _sc_gather_add(p, q, row, col, k):
    f = pl.kernel(
        functools.partial(_sc_gather_body, k * _EC),
        out_type=jax.ShapeDtypeStruct((_EC, HID), jnp.float32),
        mesh=_vmesh(),
        scratch_types=[
            pltpu.VMEM((_EW,), jnp.int32),
            pltpu.VMEM((_EW,), jnp.int32),
            pltpu.VMEM((_GNB, _G, HID), jnp.float32),
            pltpu.VMEM((_GNB, _G, HID), jnp.float32),
            pltpu.VMEM((_GNB, _G, HID), jnp.float32),
            pltpu.SemaphoreType.DMA((_GNB,)),
            pltpu.SemaphoreType.DMA((_GNB,)),
        ],
        name=f"sc_gather_{k}",
        compiler_params=_sc_compiler_params(),
    )
    return f(p, q, row, col)


_TROWS = 624
_ZB = 48
_TAIL0 = 16 * _TROWS
_TAILN = N - _TAIL0

_SS = 8
_LOOK = 4
_GS = 40
_PASSES = 5
_PE = _EW // _PASSES
_PCH = _PE // _GS
_PMAIN = (_PCH // _SS) * _SS


def _sc_segsum_body(ofs, mij_hbm, row_hbm, out_hbm, ri_v, mb_v, zb_v, acc_sh,
                    lsem, asem):
    cid = lax.axis_index("c")
    sid = lax.axis_index("s")
    base = (cid * 16 + sid) * _EW

    @pl.loop(0, _ZB)
    def _(r):
        for c0 in range(0, HID, 16):
            zb_v[r, pl.ds(c0, 16)] = jnp.zeros((16,), jnp.float32)

    @pl.loop(0, _TROWS // _ZB)
    def _(i):
        pltpu.sync_copy(zb_v, acc_sh.at[pl.ds(sid * _TROWS + i * _ZB, _ZB)])

    @pl.when(sid == 0)
    def _():
        pltpu.sync_copy(zb_v.at[pl.ds(0, _TAILN)],
                        acc_sh.at[pl.ds(_TAIL0, _TAILN)])

    plsc.subcore_barrier()

    @pl.loop(0, _PASSES)
    def _(p):
        pbase = base + p * _PE
        pltpu.sync_copy(row_hbm.at[pl.ds(ofs + pbase, _PE)], ri_v)

        def start_load(c, slot):
            pltpu.async_copy(mij_hbm.at[pl.ds(pbase + c * _GS, _GS)],
                             mb_v.at[slot], lsem.at[slot])

        def wait_load(slot):
            pltpu.make_async_copy(mij_hbm.at[pl.ds(base, _GS)],
                                  mb_v.at[slot], lsem.at[slot]).wait()

        def start_add(c, slot):
            pltpu.async_copy(mb_v.at[slot],
                             acc_sh.at[ri_v.at[pl.ds(c * _GS, _GS)]],
                             asem.at[slot], add=True)

        def wait_add(slot):
            pltpu.make_async_copy(mb_v.at[slot],
                                  acc_sh.at[ri_v.at[pl.ds(0, _GS)]],
                                  asem.at[slot]).wait()

        for c in range(_LOOK):
            start_load(c, c)

        @pl.loop(0, _PMAIN, step=_SS)
        def _(i):
            for b in range(_SS):
                c = i + b
                nslot = (b + _LOOK) % _SS
                wait_load(b)
                start_add(c, b)

                @pl.when(c >= _LOOK)
                def _():
                    wait_add(nslot)

                @pl.when(c + _LOOK < _PCH)
                def _():
                    start_load(c + _LOOK, nslot)

        for c in range(_PMAIN, _PCH):
            wait_load(c % _SS)
            start_add(c, c % _SS)
        for c in range(_PMAIN - _LOOK, _PCH):
            wait_add(c % _SS)

    plsc.subcore_barrier()

    @pl.loop(0, _TROWS // _ZB)
    def _(i):
        r0 = sid * _TROWS + i * _ZB
        pltpu.sync_copy(acc_sh.at[pl.ds(r0, _ZB)],
                        out_hbm.at[cid, pl.ds(r0, _ZB)])

    @pl.when(sid == 0)
    def _():
        pltpu.sync_copy(acc_sh.at[pl.ds(_TAIL0, _TAILN)],
                        out_hbm.at[cid, pl.ds(_TAIL0, _TAILN)])


def _sc_segsum(mij, row, k):
    f = pl.kernel(
        functools.partial(_sc_segsum_body, k * _EC),
        out_type=jax.ShapeDtypeStruct((2, N, HID), jnp.float32),
        mesh=_vmesh(),
        scratch_types=[
            pltpu.VMEM((_PE,), jnp.int32),
            pltpu.VMEM((_SS, _GS, HID), jnp.float32),
            pltpu.VMEM((_ZB, HID), jnp.float32),
            pltpu.VMEM_SHARED((N, HID), jnp.float32),
            pltpu.SemaphoreType.DMA((_SS,)),
            pltpu.SemaphoreType.DMA((_SS,)),
        ],
        name=f"sc_segsum_{k}",
    )
    return f(mij, row)


def _emlp_body0(r_ref, b1_ref, w2_ref, b2_ref, mfull_ref, m_ref):
    hid = _silu(r_ref[...] + b1_ref[...])
    v = _silu(
        jnp.dot(hid, w2_ref[...], preferred_element_type=jnp.float32)
        + b2_ref[...]
    )
    mfull_ref[...] = v
    m_ref[...] = v


def _emlp_body1(mprev_ref, r_ref, b1_ref, w2_ref, b2_ref, mfull_ref, m_ref):
    _emlp_body0(r_ref, b1_ref, w2_ref, b2_ref, mfull_ref, m_ref)


def _emlp(r, eb1, ew2, eb2, k, mij_prev):
    nblk = _EC // EBLK
    common_in = [
        pl.BlockSpec((EBLK, HID), lambda i: (i, 0)),
        pl.BlockSpec((1, HID), lambda i: (0, 0)),
        pl.BlockSpec((HID, HID), lambda i: (0, 0)),
        pl.BlockSpec((1, HID), lambda i: (0, 0)),
    ]
    out_specs = [
        pl.BlockSpec((EBLK, HID), lambda i, k=k, n=nblk: (i + k * n, 0)),
        pl.BlockSpec((EBLK, HID), lambda i: (i, 0)),
    ]
    out_shape = [
        jax.ShapeDtypeStruct((E, HID), jnp.float32),
        jax.ShapeDtypeStruct((_EC, HID), jnp.float32),
    ]
    args = (r, eb1.reshape(1, HID), ew2, eb2.reshape(1, HID))
    if k == 0:
        return pl.pallas_call(
            _emlp_body0,
            grid=(nblk,),
            in_specs=common_in,
            out_specs=out_specs,
            out_shape=out_shape,
        )(*args)
    return pl.pallas_call(
        _emlp_body1,
        grid=(nblk,),
        in_specs=[pl.BlockSpec(memory_space=pl.ANY)] + common_in,
        out_specs=out_specs,
        out_shape=out_shape,
        input_output_aliases={0: 0},
    )(mij_prev, *args)


_P_ALPHA, _P_SB, _P_CB, _P_DELTA, _P_BETA = 0, 1, 2, 3, 4
_P_QB2 = 5
_P_PHI = 8


def _node_body(params_ref, h_ref, *rest):
    (pa_refs, (qw1_ref, qw2_ref, qb1_ref, pw1_ref, pb1_ref, pw2_ref,
               pb2_ref, out_ref)) = rest[:_K], rest[_K:]
    h = h_ref[...]
    acc = pa_refs[0][0] + pa_refs[0][1]
    for _k in range(1, _K):
        acc = acc + pa_refs[_k][0] + pa_refs[_k][1]
    agg = acc * (1.0 / NORM)
    cat = jnp.concatenate([h, agg], axis=1)
    hq = _silu(jnp.dot(cat, qw1_ref[...], preferred_element_type=jnp.float32)
               + qb1_ref[...])
    qin_t = lax.dot_general(qw2_ref[...], hq,
                            (((0,), (1,)), ((), ())),
                            preferred_element_type=jnp.float32)

    alpha = params_ref[_P_ALPHA]
    sb = params_ref[_P_SB]
    cb = params_ref[_P_CB]
    delta = params_ref[_P_DELTA]
    beta = params_ref[_P_BETA]

    q = [qin_t[k:k + 1, :] + params_ref[_P_QB2 + k] for k in range(NQ)]
    sa = [jnp.sin(alpha * qk) for qk in q]
    ca = [jnp.cos(alpha * qk) for qk in q]
    _pairidx = {(0, 1): 0, (0, 2): 1, (1, 2): 2}

    zrows = []
    for k in range(NQ):
        fr, fi = None, None
        for j in range(NQ):
            if j == k:
                continue
            pi = _pairidx[(min(k, j), max(k, j))]
            cp = params_ref[_P_PHI + 2 * pi]
            sp = params_ref[_P_PHI + 2 * pi + 1]
            gr = cp
            gi = -sp * ca[j]
            if fr is None:
                fr, fi = jnp.full_like(ca[j], gr), gi
            else:
                fr, fi = fr * gr - fi * gi, fr * gi + fi * gr
        qk = q[k]
        q2 = qk * qk
        d1 = delta * (1.0 - MU * q2)
        c = beta + delta * q2
        sc, cc = jnp.sin(c), jnp.cos(c)
        sd1, cd1 = jnp.sin(d1), jnp.cos(d1)
        rx = sa[k] * fr
        ry = -sa[k] * fi
        rz = ca[k]
        mx = sc * sd1
        my = sc * cd1 * cb + cc * sb
        mz = -sc * cd1 * sb + cc * cb
        zrows.append(rx * mx + ry * my + rz * mz)
    qout_t = jnp.concatenate(zrows, axis=0)

    hp_pre = (jnp.dot(cat, pw1_ref[:2 * D, :], preferred_element_type=jnp.float32)
              + lax.dot_general(qout_t, pw1_ref[2 * D:2 * D + NQ, :],
                                (((0,), (0,)), ((), ())),
                                preferred_element_type=jnp.float32)
              + pb1_ref[...])
    hp = _silu(hp_pre)
    out_ref[...] = h + jnp.dot(hp, pw2_ref[...],
                               preferred_element_type=jnp.float32) + pb2_ref[...]


def _node_stage(h, parts, qw1, qb1, qw2, qb2, pw1, pb1, pw2, pb2,
                alpha, beta, gamma, delta, Lam):
    phi = gamma * (Lam + Lam.T) / 2.0
    params = jnp.concatenate([
        jnp.stack([alpha, jnp.sin(beta), jnp.cos(beta), delta, beta]),
        qb2,
        jnp.stack([jnp.cos(phi[0, 1]), jnp.sin(phi[0, 1]),
                   jnp.cos(phi[0, 2]), jnp.sin(phi[0, 2]),
                   jnp.cos(phi[1, 2]), jnp.sin(phi[1, 2])]),
    ]).astype(jnp.float32)
    blk = lambda shape: pl.BlockSpec(shape, lambda i: tuple(0 for _ in shape))
    return pl.pallas_call(
        _node_body,
        grid=(N // NBLK,),
        in_specs=[
            pl.BlockSpec(memory_space=pltpu.SMEM),
            pl.BlockSpec((NBLK, D), lambda i: (i, 0)),
        ] + [
            pl.BlockSpec((2, NBLK, HID), lambda i: (0, i, 0))
            for _ in range(_K)
        ] + [
            blk((2 * D, HID)),
            blk((HID, NQ)),
            blk((1, HID)),
            blk((2 * D + NQ, HID)),
            blk((1, HID)),
            blk((HID, D)),
            blk((1, D)),
        ],
        out_specs=pl.BlockSpec((NBLK, D), lambda i: (i, 0)),
        out_shape=jax.ShapeDtypeStruct((N, D), jnp.float32),
    )(params, h, *parts, qw1, qw2, qb1.reshape(1, HID), pw1,
      pb1.reshape(1, HID), pw2, pb2.reshape(1, D))


def kernel(h, edge_index, ew1, eb1, ew2, eb2, qw1, qb1, qw2, qb2,
           pw1, pb1, pw2, pb2, alpha, beta, gamma, delta, Lam):
    row = edge_index[0]
    col = edge_index[1]
    p, q = _pq(h, ew1)
    parts = []
    mij = None
    for k in range(_K):
        r_k = _sc_gather_add(p, q, row, col, k)
        mij, mij_k = _emlp(r_k, eb1, ew2, eb2, k, mij)
        parts.append(_sc_segsum(mij_k, row, k))
    h_out = _node_stage(h, parts, qw1, qb1, qw2, qb2,
                        pw1, pb1, pw2, pb2, alpha, beta, gamma, delta, Lam)
    return (h_out, mij)
